# baseline (speedup 1.0000x reference)
import jax
import jax.numpy as jnp
from jax import lax
from jax.experimental import pallas as pl
from jax.experimental.pallas import tpu as pltpu, tpu_sc as plsc

NPIX = 512 * 512
NTILE = 16
PPT = NPIX // NTILE
NVEC = PPT // 16
NLAB = 27
LPAD = 32
S1 = 4 * 512
S2 = 2 * 512 + 64
F32 = jnp.float32
I32 = jnp.int32


def _rcp_nr(t):
    i = plsc.bitcast(t, I32)
    i = 0x7EF311C3 - i
    y = plsc.bitcast(i, F32)
    for _ in range(3):
        y = y * (2.0 - t * y)
    return y


def _rsqrt_nr(c):
    i = plsc.bitcast(c, I32)
    i = 0x5F3759DF - lax.shift_right_arithmetic(i, 1)
    y = plsc.bitcast(i, F32)
    for _ in range(3):
        y = y * (1.5 - 0.5 * c * y * y)
    return y


def _zero_range(ref, base, nvec, z16):
    def zr(j, _):
        ref[pl.ds(base + j * 16, 16)] = z16
        return 0
    lax.fori_loop(0, nvec, zr, 0)


def _acc_range(dst, src, nvec):
    def ar(j, _):
        sl = pl.ds(j * 16, 16)
        dst[sl] = dst[sl] + src[sl]
        return 0
    lax.fori_loop(0, nvec, ar, 0)


def _sc_body(pred_hbm, tgt_hbm, nb_hbm, out_hbm,
             px, tx, stats, varsep, tmp,
             cntv, m0v, m1v, m2v,
             nbv, outv,
             sh1, sh1g, sh2):
    c = lax.axis_index("c")
    s = lax.axis_index("s")
    lanes = lax.iota(I32, 16)
    lanesf = lanes.astype(F32)
    z16 = jnp.zeros((16,), F32)
    ones16 = jnp.ones((16,), F32)

    pltpu.sync_copy(nb_hbm, nbv)

    def one_batch(bl, _):
        b = c * 2 + bl
        base = s * PPT

        _zero_range(stats, 0, S1 // 16, z16)
        _zero_range(varsep, 0, S2 // 16, z16)

        for ch in range(3):
            hoff = (b * 3 + ch) * NPIX + base
            pltpu.sync_copy(pred_hbm.at[pl.ds(hoff, PPT)],
                            px.at[pl.ds(ch * PPT, PPT)])
            pltpu.sync_copy(tgt_hbm.at[pl.ds(hoff, PPT)],
                            tx.at[pl.ds(ch * PPT, PPT)])

        def p1(i, _):
            sl0 = pl.ds(i * 16, 16)
            sl1 = pl.ds(PPT + i * 16, 16)
            sl2 = pl.ds(2 * PPT + i * 16, 16)
            labv = tx[sl0] * 9 + tx[sl1] * 3 + tx[sl2]
            addr = lanes * LPAD + labv
            plsc.addupdate_scatter(stats, [addr], ones16)
            plsc.addupdate_scatter(stats, [addr + 512], px[sl0])
            plsc.addupdate_scatter(stats, [addr + 1024], px[sl1])
            plsc.addupdate_scatter(stats, [addr + 1536], px[sl2])
            return 0
        lax.fori_loop(0, NVEC, p1, 0)

        pltpu.sync_copy(stats, sh1.at[pl.ds(s * S1, S1)])
        plsc.subcore_barrier()

        @pl.when(s == 0)
        def _():
            for t in range(1, NTILE):
                pltpu.sync_copy(sh1.at[pl.ds(t * S1, S1)], tmp)
                _acc_range(stats, tmp, S1 // 16)
            pltpu.sync_copy(stats, sh1g)
        plsc.subcore_barrier()
        pltpu.sync_copy(sh1g, stats)

        for k in range(2):
            ksl = pl.ds(k * 16, 16)
            cnt = z16
            s0 = z16
            s1 = z16
            s2 = z16
            for l in range(16):
                off = l * LPAD + k * 16
                cnt = cnt + stats[pl.ds(off, 16)]
                s0 = s0 + stats[pl.ds(512 + off, 16)]
                s1 = s1 + stats[pl.ds(1024 + off, 16)]
                s2 = s2 + stats[pl.ds(1536 + off, 16)]
            cm = jnp.maximum(cnt, 1.0)
            m0 = s0 / cm
            m1 = s1 / cm
            m2 = s2 / cm
            if k == 0:
                bgz = jnp.where(lanesf == 0.0, 0.0, 1.0)
                m0 = m0 * bgz
                m1 = m1 * bgz
                m2 = m2 * bgz
            cntv[ksl] = cnt
            m0v[ksl] = m0
            m1v[ksl] = m1
            m2v[ksl] = m2

        def p2(i, _):
            sl0 = pl.ds(i * 16, 16)
            sl1 = pl.ds(PPT + i * 16, 16)
            sl2 = pl.ds(2 * PPT + i * 16, 16)
            labv = tx[sl0] * 9 + tx[sl1] * 3 + tx[sl2]
            addr = lanes * LPAD + labv
            d0 = px[sl0] - plsc.load_gather(m0v, [labv])
            d1 = px[sl1] - plsc.load_gather(m1v, [labv])
            d2 = px[sl2] - plsc.load_gather(m2v, [labv])
            h = z16
            for d in (d0, d1, d2):
                ad = jnp.abs(d)
                h = h + jnp.where(ad <= 1.0, 0.5 * d * d, ad - 0.5)
            plsc.addupdate_scatter(varsep, [addr], h)
            so = 300.0 / (1.0 + (d0 * d0 + d1 * d1 + d2 * d2))
            plsc.addupdate_scatter(varsep, [addr + 512], so)
            return 0
        lax.fori_loop(0, NVEC, p2, 0)

        mck = [(m0v[pl.ds(k * 16, 16)], m1v[pl.ds(k * 16, 16)],
                m2v[pl.ds(k * 16, 16)]) for k in range(2)]
        a0c = [-2.0 * m for (m, _, _) in mck]
        a1c = [-2.0 * m for (_, m, _) in mck]
        a2c = [-2.0 * m for (_, _, m) in mck]
        ccc = [1.0 + m0 * m0 + m1 * m1 + m2 * m2 for (m0, m1, m2) in mck]
        GBLK = 4
        UNR = 2
        for g0 in range(1, NLAB, GBLK):
            gs = list(range(g0, min(g0 + GBLK, NLAB)))
            co = [(a0c[g // 16][g % 16], a1c[g // 16][g % 16],
                   a2c[g // 16][g % 16], ccc[g // 16][g % 16]) for g in gs]

            def pg(i, accs, co=co):
                accs = list(accs)
                for u in range(UNR):
                    sl = pl.ds(i * (16 * UNR) + u * 16, 16)
                    x0 = px[sl]
                    x1 = px[pl.ds(PPT + i * (16 * UNR) + u * 16, 16)]
                    x2 = px[pl.ds(2 * PPT + i * (16 * UNR) + u * 16, 16)]
                    n = x0 * x0 + x1 * x1 + x2 * x2
                    for j, (a0, a1, a2, cg) in enumerate(co):
                        t = (n + cg) + (a0 * x0 + a1 * x1 + a2 * x2)
                        accs[j] = accs[j] + 300.0 * _rcp_nr(t)
                return tuple(accs)
            accs = lax.fori_loop(0, NVEC // UNR, pg, (z16,) * len(gs))
            for g, acc in zip(gs, accs):
                sg = jnp.sum(acc)
                off = 1024 + (g // 16) * 16
                contrib = jnp.where(lanes == (g % 16), sg, 0.0)
                varsep[pl.ds(off, 16)] = varsep[pl.ds(off, 16)] + contrib

        pltpu.sync_copy(varsep, sh2.at[pl.ds(s * S2, S2)])
        plsc.subcore_barrier()

        @pl.when(s == 0)
        def _():
            for t in range(1, NTILE):
                pltpu.sync_copy(sh2.at[pl.ds(t * S2, S2)], tmp.at[pl.ds(0, S2)])
                _acc_range(varsep, tmp, S2 // 16)

            alv = plsc.load_gather(nbv, [jnp.full((16,), 0, I32) + b])
            lv = z16
            ctv = z16
            presk = [None, None]
            for k in range(2):
                ksl = pl.ds(k * 16, 16)
                varsum = z16
                sepown = z16
                for l in range(16):
                    off = l * LPAD + k * 16
                    varsum = varsum + varsep[pl.ds(off, 16)]
                    sepown = sepown + varsep[pl.ds(512 + off, 16)]
                cv = cntv[ksl]
                cm = jnp.maximum(cv, 1.0)
                mpos = jnp.where(cv > 0.0, 1.0, 0.0)
                if k == 0:
                    pres = mpos * jnp.where(lanesf == 0.0, alv, 1.0)
                    gz = jnp.where(lanesf > 0.0, 1.0, 0.0)
                else:
                    pres = mpos
                    gz = ones16
                presk[k] = pres
                termA = pres * varsum / (3.0 * cm)
                other = float(NPIX) - cv
                op = jnp.where(other > 0.0, 1.0, 0.0)
                wb = 10.0 * _rsqrt_nr(cm)
                sepall = varsep[pl.ds(1024 + k * 16, 16)]
                termB = (mpos * gz * op * (sepall - sepown)
                         / jnp.maximum(other, 1.0) * wb)
                lv = lv + termA + termB
                ctv = ctv + pres

            mfk = [(m0v[pl.ds(k * 16, 16)], m1v[pl.ds(k * 16, 16)],
                    m2v[pl.ds(k * 16, 16)]) for k in range(2)]
            pacc = z16
            nacc = z16
            for i in range(NLAB):
                mi = mfk[i // 16]
                pi = presk[i // 16][i % 16]
                m0i = mi[0][i % 16]
                m1i = mi[1][i % 16]
                m2i = mi[2][i % 16]
                for k in range(2):
                    jv = lanesf + float(k * 16)
                    d0 = mfk[k][0] - m0i
                    d1 = mfk[k][1] - m1i
                    d2 = mfk[k][2] - m2i
                    sq = d0 * d0 + d1 * d1 + d2 * d2
                    mk = jnp.where(jv > float(i), 1.0, 0.0) * presk[k] * pi
                    pacc = pacc + 300.0 / (sq + 1.0) * mk
                    nacc = nacc + mk
            spn = z16 + jnp.sum(nacc)
            spt = (z16 + jnp.sum(pacc)) / jnp.maximum(spn, 1.0)
            lossv = (z16 + jnp.sum(lv)) + jnp.where(spn > 0.0, spt, z16)
            outvec = lossv / jnp.maximum(z16 + jnp.sum(ctv), 1.0)
            outv[pl.ds(0, 16)] = jnp.where(lanes == 0, outvec, z16)
            pltpu.sync_copy(outv, out_hbm.at[pl.ds(b * 16, 16)])
        plsc.subcore_barrier()
        return 0

    lax.fori_loop(0, 2, one_batch, 0)


@jax.jit
def _distance_loss(pred, tgt, nb):
    mesh = plsc.VectorSubcoreMesh(core_axis_name="c", subcore_axis_name="s",
                                  num_cores=2, num_subcores=16)
    run = pl.kernel(
        _sc_body,
        out_type=jax.ShapeDtypeStruct((64,), F32),
        mesh=mesh,
        compiler_params=pltpu.CompilerParams(
            needs_layout_passes=False, use_tc_tiling_on_sc=False),
        scratch_types=[
            pltpu.VMEM((3 * PPT,), F32),
            pltpu.VMEM((3 * PPT,), I32),
            pltpu.VMEM((S1,), F32),
            pltpu.VMEM((S2,), F32),
            pltpu.VMEM((S1,), F32),
            pltpu.VMEM((LPAD,), F32),
            pltpu.VMEM((LPAD,), F32),
            pltpu.VMEM((LPAD,), F32),
            pltpu.VMEM((LPAD,), F32),
            pltpu.VMEM((16,), F32),
            pltpu.VMEM((16,), F32),
            pltpu.VMEM_SHARED((NTILE * S1,), F32),
            pltpu.VMEM_SHARED((S1,), F32),
            pltpu.VMEM_SHARED((NTILE * S2,), F32),
        ],
    )
    return run(pred, tgt, nb)


def kernel(prediction, target, no_bg):
    B, C, H, W = prediction.shape
    pred = prediction.astype(F32).reshape(-1)
    tgt = target.astype(I32).reshape(-1)
    nb = jnp.zeros((16,), F32).at[:B].set(1.0 - no_bg.astype(F32))
    out = _distance_loss(pred, tgt, nb)
    return jnp.sum(out.reshape(B, 16)[:, 0]) / float(B)


# speedup vs baseline: 1.2509x; 1.2509x over previous
import jax
import jax.numpy as jnp
from jax import lax
from jax.experimental import pallas as pl
from jax.experimental.pallas import tpu as pltpu, tpu_sc as plsc

NPIX = 512 * 512
NTILE = 16
PPT = NPIX // NTILE
NVEC = PPT // 16
NLAB = 27
LPAD = 32
S1 = 4 * 512
S2 = 2 * 512 + 64
F32 = jnp.float32
I32 = jnp.int32


def _rcp_nr(t):
    i = plsc.bitcast(t, I32)
    i = 0x7EF311C3 - i
    y = plsc.bitcast(i, F32)
    for _ in range(3):
        y = y * (2.0 - t * y)
    return y


def _rsqrt_nr(c):
    i = plsc.bitcast(c, I32)
    i = 0x5F3759DF - lax.shift_right_arithmetic(i, 1)
    y = plsc.bitcast(i, F32)
    for _ in range(3):
        y = y * (1.5 - 0.5 * c * y * y)
    return y


def _zero_range(ref, base, nvec, z16):
    def zr(j, _):
        ref[pl.ds(base + j * 16, 16)] = z16
        return 0
    lax.fori_loop(0, nvec, zr, 0)


def _acc_range(dst, src, nvec):
    def ar(j, _):
        sl = pl.ds(j * 16, 16)
        dst[sl] = dst[sl] + src[sl]
        return 0
    lax.fori_loop(0, nvec, ar, 0)


def _sc_body(pred_hbm, tgt_hbm, nb_hbm, out_hbm,
             px, tx, stats, varsep, tmp,
             cntv, m0v, m1v, m2v,
             nbv, outv,
             sh1, sh1g, sh2):
    c = lax.axis_index("c")
    s = lax.axis_index("s")
    lanes = lax.iota(I32, 16)
    lanesf = lanes.astype(F32)
    z16 = jnp.zeros((16,), F32)
    ones16 = jnp.ones((16,), F32)

    pltpu.sync_copy(nb_hbm, nbv)

    def one_batch(bl, _):
        b = c * 2 + bl
        base = s * PPT

        _zero_range(stats, 0, S1 // 16, z16)
        _zero_range(varsep, 0, S2 // 16, z16)

        for ch in range(3):
            hoff = (b * 3 + ch) * NPIX + base
            pltpu.sync_copy(pred_hbm.at[pl.ds(hoff, PPT)],
                            px.at[pl.ds(ch * PPT, PPT)])
            pltpu.sync_copy(tgt_hbm.at[pl.ds(hoff, PPT)],
                            tx.at[pl.ds(ch * PPT, PPT)])

        def p1(i, _):
            sl0 = pl.ds(i * 16, 16)
            sl1 = pl.ds(PPT + i * 16, 16)
            sl2 = pl.ds(2 * PPT + i * 16, 16)
            labv = tx[sl0] * 9 + tx[sl1] * 3 + tx[sl2]
            addr = lanes * LPAD + labv
            plsc.addupdate_scatter(stats, [addr], ones16)
            plsc.addupdate_scatter(stats, [addr + 512], px[sl0])
            plsc.addupdate_scatter(stats, [addr + 1024], px[sl1])
            plsc.addupdate_scatter(stats, [addr + 1536], px[sl2])
            return 0
        lax.fori_loop(0, NVEC, p1, 0)

        pltpu.sync_copy(stats, sh1.at[pl.ds(s * S1, S1)])
        plsc.subcore_barrier()

        @pl.when(s == 0)
        def _():
            for t in range(1, NTILE):
                pltpu.sync_copy(sh1.at[pl.ds(t * S1, S1)], tmp)
                _acc_range(stats, tmp, S1 // 16)
            pltpu.sync_copy(stats, sh1g)
        plsc.subcore_barrier()
        pltpu.sync_copy(sh1g, stats)

        for k in range(2):
            ksl = pl.ds(k * 16, 16)
            cnt = z16
            s0 = z16
            s1 = z16
            s2 = z16
            for l in range(16):
                off = l * LPAD + k * 16
                cnt = cnt + stats[pl.ds(off, 16)]
                s0 = s0 + stats[pl.ds(512 + off, 16)]
                s1 = s1 + stats[pl.ds(1024 + off, 16)]
                s2 = s2 + stats[pl.ds(1536 + off, 16)]
            cm = jnp.maximum(cnt, 1.0)
            m0 = s0 / cm
            m1 = s1 / cm
            m2 = s2 / cm
            if k == 0:
                bgz = jnp.where(lanesf == 0.0, 0.0, 1.0)
                m0 = m0 * bgz
                m1 = m1 * bgz
                m2 = m2 * bgz
            cntv[ksl] = cnt
            m0v[ksl] = m0
            m1v[ksl] = m1
            m2v[ksl] = m2

        def p2(i, _):
            sl0 = pl.ds(i * 16, 16)
            sl1 = pl.ds(PPT + i * 16, 16)
            sl2 = pl.ds(2 * PPT + i * 16, 16)
            labv = tx[sl0] * 9 + tx[sl1] * 3 + tx[sl2]
            addr = lanes * LPAD + labv
            d0 = px[sl0] - plsc.load_gather(m0v, [labv])
            d1 = px[sl1] - plsc.load_gather(m1v, [labv])
            d2 = px[sl2] - plsc.load_gather(m2v, [labv])
            h = z16
            for d in (d0, d1, d2):
                ad = jnp.abs(d)
                h = h + jnp.where(ad <= 1.0, 0.5 * d * d, ad - 0.5)
            plsc.addupdate_scatter(varsep, [addr], h)
            so = 300.0 / (1.0 + (d0 * d0 + d1 * d1 + d2 * d2))
            plsc.addupdate_scatter(varsep, [addr + 512], so)
            return 0
        lax.fori_loop(0, NVEC, p2, 0)

        mck = [(m0v[pl.ds(k * 16, 16)], m1v[pl.ds(k * 16, 16)],
                m2v[pl.ds(k * 16, 16)]) for k in range(2)]
        a0c = [-2.0 * m for (m, _, _) in mck]
        a1c = [-2.0 * m for (_, m, _) in mck]
        a2c = [-2.0 * m for (_, _, m) in mck]
        ccc = [1.0 + m0 * m0 + m1 * m1 + m2 * m2 for (m0, m1, m2) in mck]
        GBLK = 4
        for g0 in range(1, NLAB, GBLK):
            gs = list(range(g0, min(g0 + GBLK, NLAB)))
            co = [(a0c[g // 16][g % 16], a1c[g // 16][g % 16],
                   a2c[g // 16][g % 16], ccc[g // 16][g % 16]) for g in gs]

            def pg(i, accs, co=co):
                accs = list(accs)
                x0 = px[pl.ds(i * 16, 16)]
                x1 = px[pl.ds(PPT + i * 16, 16)]
                x2 = px[pl.ds(2 * PPT + i * 16, 16)]
                n = x0 * x0 + x1 * x1 + x2 * x2
                for j, (a0, a1, a2, cg) in enumerate(co):
                    t = (n + cg) + (a0 * x0 + a1 * x1 + a2 * x2)
                    accs[j] = accs[j] + 300.0 / t
                return tuple(accs)
            accs = plsc.parallel_loop(0, NVEC, 1, unroll=4,
                                      carry=(z16,) * len(gs))(pg)
            for g, acc in zip(gs, accs):
                sg = jnp.sum(acc)
                off = 1024 + (g // 16) * 16
                contrib = jnp.where(lanes == (g % 16), sg, 0.0)
                varsep[pl.ds(off, 16)] = varsep[pl.ds(off, 16)] + contrib

        pltpu.sync_copy(varsep, sh2.at[pl.ds(s * S2, S2)])
        plsc.subcore_barrier()

        @pl.when(s == 0)
        def _():
            for t in range(1, NTILE):
                pltpu.sync_copy(sh2.at[pl.ds(t * S2, S2)], tmp.at[pl.ds(0, S2)])
                _acc_range(varsep, tmp, S2 // 16)

            alv = plsc.load_gather(nbv, [jnp.full((16,), 0, I32) + b])
            lv = z16
            ctv = z16
            presk = [None, None]
            for k in range(2):
                ksl = pl.ds(k * 16, 16)
                varsum = z16
                sepown = z16
                for l in range(16):
                    off = l * LPAD + k * 16
                    varsum = varsum + varsep[pl.ds(off, 16)]
                    sepown = sepown + varsep[pl.ds(512 + off, 16)]
                cv = cntv[ksl]
                cm = jnp.maximum(cv, 1.0)
                mpos = jnp.where(cv > 0.0, 1.0, 0.0)
                if k == 0:
                    pres = mpos * jnp.where(lanesf == 0.0, alv, 1.0)
                    gz = jnp.where(lanesf > 0.0, 1.0, 0.0)
                else:
                    pres = mpos
                    gz = ones16
                presk[k] = pres
                termA = pres * varsum / (3.0 * cm)
                other = float(NPIX) - cv
                op = jnp.where(other > 0.0, 1.0, 0.0)
                wb = 10.0 * _rsqrt_nr(cm)
                sepall = varsep[pl.ds(1024 + k * 16, 16)]
                termB = (mpos * gz * op * (sepall - sepown)
                         / jnp.maximum(other, 1.0) * wb)
                lv = lv + termA + termB
                ctv = ctv + pres

            mfk = [(m0v[pl.ds(k * 16, 16)], m1v[pl.ds(k * 16, 16)],
                    m2v[pl.ds(k * 16, 16)]) for k in range(2)]
            pacc = z16
            nacc = z16
            for i in range(NLAB):
                mi = mfk[i // 16]
                pi = presk[i // 16][i % 16]
                m0i = mi[0][i % 16]
                m1i = mi[1][i % 16]
                m2i = mi[2][i % 16]
                for k in range(2):
                    jv = lanesf + float(k * 16)
                    d0 = mfk[k][0] - m0i
                    d1 = mfk[k][1] - m1i
                    d2 = mfk[k][2] - m2i
                    sq = d0 * d0 + d1 * d1 + d2 * d2
                    mk = jnp.where(jv > float(i), 1.0, 0.0) * presk[k] * pi
                    pacc = pacc + 300.0 / (sq + 1.0) * mk
                    nacc = nacc + mk
            spn = z16 + jnp.sum(nacc)
            spt = (z16 + jnp.sum(pacc)) / jnp.maximum(spn, 1.0)
            lossv = (z16 + jnp.sum(lv)) + jnp.where(spn > 0.0, spt, z16)
            outvec = lossv / jnp.maximum(z16 + jnp.sum(ctv), 1.0)
            outv[pl.ds(0, 16)] = jnp.where(lanes == 0, outvec, z16)
            pltpu.sync_copy(outv, out_hbm.at[pl.ds(b * 16, 16)])
        plsc.subcore_barrier()
        return 0

    lax.fori_loop(0, 2, one_batch, 0)


@jax.jit
def _distance_loss(pred, tgt, nb):
    mesh = plsc.VectorSubcoreMesh(core_axis_name="c", subcore_axis_name="s",
                                  num_cores=2, num_subcores=16)
    run = pl.kernel(
        _sc_body,
        out_type=jax.ShapeDtypeStruct((64,), F32),
        mesh=mesh,
        compiler_params=pltpu.CompilerParams(
            needs_layout_passes=False, use_tc_tiling_on_sc=False),
        scratch_types=[
            pltpu.VMEM((3 * PPT,), F32),
            pltpu.VMEM((3 * PPT,), I32),
            pltpu.VMEM((S1,), F32),
            pltpu.VMEM((S2,), F32),
            pltpu.VMEM((S1,), F32),
            pltpu.VMEM((LPAD,), F32),
            pltpu.VMEM((LPAD,), F32),
            pltpu.VMEM((LPAD,), F32),
            pltpu.VMEM((LPAD,), F32),
            pltpu.VMEM((16,), F32),
            pltpu.VMEM((16,), F32),
            pltpu.VMEM_SHARED((NTILE * S1,), F32),
            pltpu.VMEM_SHARED((S1,), F32),
            pltpu.VMEM_SHARED((NTILE * S2,), F32),
        ],
    )
    return run(pred, tgt, nb)


def kernel(prediction, target, no_bg):
    B, C, H, W = prediction.shape
    pred = prediction.astype(F32).reshape(-1)
    tgt = target.astype(I32).reshape(-1)
    nb = jnp.zeros((16,), F32).at[:B].set(1.0 - no_bg.astype(F32))
    out = _distance_loss(pred, tgt, nb)
    return jnp.sum(out.reshape(B, 16)[:, 0]) / float(B)


# speedup vs baseline: 1.7931x; 1.4334x over previous
import jax
import jax.numpy as jnp
from jax import lax
from jax.experimental import pallas as pl
from jax.experimental.pallas import tpu as pltpu, tpu_sc as plsc

NPIX = 512 * 512
NTILE = 16
PPT = NPIX // NTILE
NVEC = PPT // 16
NLAB = 27
LPAD = 32
S1 = 4 * 512
S2 = 1280
F32 = jnp.float32
I32 = jnp.int32


def _rcp_nr(t):
    i = plsc.bitcast(t, I32)
    i = 0x7EF311C3 - i
    y = plsc.bitcast(i, F32)
    for _ in range(3):
        y = y * (2.0 - t * y)
    return y


def _rsqrt_nr(c):
    i = plsc.bitcast(c, I32)
    i = 0x5F3759DF - lax.shift_right_arithmetic(i, 1)
    y = plsc.bitcast(i, F32)
    for _ in range(3):
        y = y * (1.5 - 0.5 * c * y * y)
    return y


def _zero_range(ref, base, nvec, z16):
    def zr(j, _):
        ref[pl.ds(base + j * 16, 16)] = z16
        return 0
    lax.fori_loop(0, nvec, zr, 0)


def _sc_body(pred_hbm, tgt_hbm, nb_hbm, out_hbm,
             px, tx, stats, varsep, tmp, red,
             cntv, m0v, m1v, m2v,
             nbv, outv,
             sh1, sh1g, sh2, sh2g):
    c = lax.axis_index("c")
    s = lax.axis_index("s")
    lanes = lax.iota(I32, 16)
    lanesf = lanes.astype(F32)
    z16 = jnp.zeros((16,), F32)
    ones16 = jnp.ones((16,), F32)

    pltpu.sync_copy(nb_hbm, nbv)

    def one_batch(bl, _):
        b = c * 2 + bl
        base = s * PPT

        _zero_range(stats, 0, S1 // 16, z16)
        _zero_range(varsep, 0, S2 // 16, z16)

        for ch in range(3):
            hoff = (b * 3 + ch) * NPIX + base
            pltpu.sync_copy(pred_hbm.at[pl.ds(hoff, PPT)],
                            px.at[pl.ds(ch * PPT, PPT)])
            pltpu.sync_copy(tgt_hbm.at[pl.ds(hoff, PPT)],
                            tx.at[pl.ds(ch * PPT, PPT)])

        def p1(i):
            sl0 = pl.ds(i * 16, 16)
            sl1 = pl.ds(PPT + i * 16, 16)
            sl2 = pl.ds(2 * PPT + i * 16, 16)
            labv = tx[sl0] * 9 + tx[sl1] * 3 + tx[sl2]
            addr = lanes * LPAD + labv
            plsc.addupdate_scatter(stats, [addr], ones16)
            plsc.addupdate_scatter(stats, [addr + 512], px[sl0])
            plsc.addupdate_scatter(stats, [addr + 1024], px[sl1])
            plsc.addupdate_scatter(stats, [addr + 1536], px[sl2])
        plsc.parallel_loop(0, NVEC, 1, unroll=4)(p1)

        pltpu.sync_copy(stats, sh1.at[pl.ds(s * S1, S1)])
        plsc.subcore_barrier()
        myco = s * 128
        for ts in range(NTILE):
            pltpu.sync_copy(sh1.at[pl.ds(ts * S1 + myco, 128)],
                            tmp.at[pl.ds(ts * 128, 128)])
        for j in range(8):
            acc = z16
            for ts in range(NTILE):
                acc = acc + tmp[pl.ds(ts * 128 + j * 16, 16)]
            red[pl.ds(j * 16, 16)] = acc
        pltpu.sync_copy(red.at[pl.ds(0, 128)], sh1g.at[pl.ds(s * 128, 128)])
        plsc.subcore_barrier()
        pltpu.sync_copy(sh1g, stats)

        for k in range(2):
            ksl = pl.ds(k * 16, 16)
            cnt = z16
            s0 = z16
            s1 = z16
            s2 = z16
            for l in range(16):
                off = l * LPAD + k * 16
                cnt = cnt + stats[pl.ds(off, 16)]
                s0 = s0 + stats[pl.ds(512 + off, 16)]
                s1 = s1 + stats[pl.ds(1024 + off, 16)]
                s2 = s2 + stats[pl.ds(1536 + off, 16)]
            cm = jnp.maximum(cnt, 1.0)
            m0 = s0 / cm
            m1 = s1 / cm
            m2 = s2 / cm
            if k == 0:
                bgz = jnp.where(lanesf == 0.0, 0.0, 1.0)
                m0 = m0 * bgz
                m1 = m1 * bgz
                m2 = m2 * bgz
            cntv[ksl] = cnt
            m0v[ksl] = m0
            m1v[ksl] = m1
            m2v[ksl] = m2

        def p2(i):
            sl0 = pl.ds(i * 16, 16)
            sl1 = pl.ds(PPT + i * 16, 16)
            sl2 = pl.ds(2 * PPT + i * 16, 16)
            labv = tx[sl0] * 9 + tx[sl1] * 3 + tx[sl2]
            addr = lanes * LPAD + labv
            d0 = px[sl0] - plsc.load_gather(m0v, [labv])
            d1 = px[sl1] - plsc.load_gather(m1v, [labv])
            d2 = px[sl2] - plsc.load_gather(m2v, [labv])
            h = z16
            for d in (d0, d1, d2):
                ad = jnp.abs(d)
                h = h + jnp.where(ad <= 1.0, 0.5 * d * d, ad - 0.5)
            plsc.addupdate_scatter(varsep, [addr], h)
            so = 300.0 / (1.0 + (d0 * d0 + d1 * d1 + d2 * d2))
            plsc.addupdate_scatter(varsep, [addr + 512], so)
        plsc.parallel_loop(0, NVEC, 1, unroll=4)(p2)

        mck = [(m0v[pl.ds(k * 16, 16)], m1v[pl.ds(k * 16, 16)],
                m2v[pl.ds(k * 16, 16)]) for k in range(2)]
        a0c = [-2.0 * m for (m, _, _) in mck]
        a1c = [-2.0 * m for (_, m, _) in mck]
        a2c = [-2.0 * m for (_, _, m) in mck]
        ccc = [1.0 + m0 * m0 + m1 * m1 + m2 * m2 for (m0, m1, m2) in mck]
        GBLK = 4
        for g0 in range(1, NLAB, GBLK):
            gs = list(range(g0, min(g0 + GBLK, NLAB)))
            co = [(a0c[g // 16][g % 16], a1c[g // 16][g % 16],
                   a2c[g // 16][g % 16], ccc[g // 16][g % 16]) for g in gs]

            def pg(i, accs, co=co):
                accs = list(accs)
                x0 = px[pl.ds(i * 16, 16)]
                x1 = px[pl.ds(PPT + i * 16, 16)]
                x2 = px[pl.ds(2 * PPT + i * 16, 16)]
                n = x0 * x0 + x1 * x1 + x2 * x2
                for j, (a0, a1, a2, cg) in enumerate(co):
                    t = (n + cg) + (a0 * x0 + a1 * x1 + a2 * x2)
                    accs[j] = accs[j] + 300.0 / t
                return tuple(accs)
            accs = lax.fori_loop(0, NVEC, pg, (z16,) * len(gs))
            for g, acc in zip(gs, accs):
                sg = jnp.sum(acc)
                off = 1024 + (g // 16) * 16
                contrib = jnp.where(lanes == (g % 16), sg, 0.0)
                varsep[pl.ds(off, 16)] = varsep[pl.ds(off, 16)] + contrib

        pltpu.sync_copy(varsep, sh2.at[pl.ds(s * S2, S2)])
        plsc.subcore_barrier()
        myc2 = s * 80
        for ts in range(NTILE):
            pltpu.sync_copy(sh2.at[pl.ds(ts * S2 + myc2, 80)],
                            tmp.at[pl.ds(ts * 80, 80)])
        for j in range(5):
            acc = z16
            for ts in range(NTILE):
                acc = acc + tmp[pl.ds(ts * 80 + j * 16, 16)]
            red[pl.ds(j * 16, 16)] = acc
        pltpu.sync_copy(red.at[pl.ds(0, 80)], sh2g.at[pl.ds(s * 80, 80)])
        plsc.subcore_barrier()

        @pl.when(s == 0)
        def _():
            pltpu.sync_copy(sh2g, varsep.at[pl.ds(0, S2)])
            alv = plsc.load_gather(nbv, [jnp.full((16,), 0, I32) + b])
            lv = z16
            ctv = z16
            presk = [None, None]
            for k in range(2):
                ksl = pl.ds(k * 16, 16)
                varsum = z16
                sepown = z16
                for l in range(16):
                    off = l * LPAD + k * 16
                    varsum = varsum + varsep[pl.ds(off, 16)]
                    sepown = sepown + varsep[pl.ds(512 + off, 16)]
                cv = cntv[ksl]
                cm = jnp.maximum(cv, 1.0)
                mpos = jnp.where(cv > 0.0, 1.0, 0.0)
                if k == 0:
                    pres = mpos * jnp.where(lanesf == 0.0, alv, 1.0)
                    gz = jnp.where(lanesf > 0.0, 1.0, 0.0)
                else:
                    pres = mpos
                    gz = ones16
                presk[k] = pres
                termA = pres * varsum / (3.0 * cm)
                other = float(NPIX) - cv
                op = jnp.where(other > 0.0, 1.0, 0.0)
                wb = 10.0 * _rsqrt_nr(cm)
                sepall = varsep[pl.ds(1024 + k * 16, 16)]
                termB = (mpos * gz * op * (sepall - sepown)
                         / jnp.maximum(other, 1.0) * wb)
                lv = lv + termA + termB
                ctv = ctv + pres

            mfk = [(m0v[pl.ds(k * 16, 16)], m1v[pl.ds(k * 16, 16)],
                    m2v[pl.ds(k * 16, 16)]) for k in range(2)]
            pacc = z16
            nacc = z16
            for i in range(NLAB):
                mi = mfk[i // 16]
                pi = presk[i // 16][i % 16]
                m0i = mi[0][i % 16]
                m1i = mi[1][i % 16]
                m2i = mi[2][i % 16]
                for k in range(2):
                    jv = lanesf + float(k * 16)
                    d0 = mfk[k][0] - m0i
                    d1 = mfk[k][1] - m1i
                    d2 = mfk[k][2] - m2i
                    sq = d0 * d0 + d1 * d1 + d2 * d2
                    mk = jnp.where(jv > float(i), 1.0, 0.0) * presk[k] * pi
                    pacc = pacc + 300.0 / (sq + 1.0) * mk
                    nacc = nacc + mk
            spn = z16 + jnp.sum(nacc)
            spt = (z16 + jnp.sum(pacc)) / jnp.maximum(spn, 1.0)
            lossv = (z16 + jnp.sum(lv)) + jnp.where(spn > 0.0, spt, z16)
            outvec = lossv / jnp.maximum(z16 + jnp.sum(ctv), 1.0)
            outv[pl.ds(0, 16)] = jnp.where(lanes == 0, outvec, z16)
            pltpu.sync_copy(outv, out_hbm.at[pl.ds(b * 16, 16)])
        plsc.subcore_barrier()
        return 0

    lax.fori_loop(0, 2, one_batch, 0)


@jax.jit
def _distance_loss(pred, tgt, nb):
    mesh = plsc.VectorSubcoreMesh(core_axis_name="c", subcore_axis_name="s",
                                  num_cores=2, num_subcores=16)
    run = pl.kernel(
        _sc_body,
        out_type=jax.ShapeDtypeStruct((64,), F32),
        mesh=mesh,
        compiler_params=pltpu.CompilerParams(
            needs_layout_passes=False, use_tc_tiling_on_sc=False),
        scratch_types=[
            pltpu.VMEM((3 * PPT,), F32),
            pltpu.VMEM((3 * PPT,), I32),
            pltpu.VMEM((S1,), F32),
            pltpu.VMEM((S2,), F32),
            pltpu.VMEM((S1,), F32),
            pltpu.VMEM((128,), F32),
            pltpu.VMEM((LPAD,), F32),
            pltpu.VMEM((LPAD,), F32),
            pltpu.VMEM((LPAD,), F32),
            pltpu.VMEM((LPAD,), F32),
            pltpu.VMEM((16,), F32),
            pltpu.VMEM((16,), F32),
            pltpu.VMEM_SHARED((NTILE * S1,), F32),
            pltpu.VMEM_SHARED((S1,), F32),
            pltpu.VMEM_SHARED((NTILE * S2,), F32),
            pltpu.VMEM_SHARED((S2,), F32),
        ],
    )
    return run(pred, tgt, nb)


def kernel(prediction, target, no_bg):
    B, C, H, W = prediction.shape
    pred = prediction.astype(F32).reshape(-1)
    tgt = target.astype(I32).reshape(-1)
    nb = jnp.zeros((16,), F32).at[:B].set(1.0 - no_bg.astype(F32))
    out = _distance_loss(pred, tgt, nb)
    return jnp.sum(out.reshape(B, 16)[:, 0]) / float(B)


# speedup vs baseline: 1.8407x; 1.0265x over previous
import jax
import jax.numpy as jnp
from jax import lax
from jax.experimental import pallas as pl
from jax.experimental.pallas import tpu as pltpu, tpu_sc as plsc

NPIX = 512 * 512
NTILE = 16
PPT = NPIX // NTILE
NVEC = PPT // 16
NLAB = 27
LPAD = 32
S1 = 4 * 512
S2 = 1280
F32 = jnp.float32
I32 = jnp.int32


def _rcp_nr(t):
    i = plsc.bitcast(t, I32)
    i = 0x7EF311C3 - i
    y = plsc.bitcast(i, F32)
    for _ in range(3):
        y = y * (2.0 - t * y)
    return y


def _rsqrt_nr(c):
    i = plsc.bitcast(c, I32)
    i = 0x5F3759DF - lax.shift_right_arithmetic(i, 1)
    y = plsc.bitcast(i, F32)
    for _ in range(3):
        y = y * (1.5 - 0.5 * c * y * y)
    return y


def _zero_range(ref, base, nvec, z16):
    def zr(j, _):
        ref[pl.ds(base + j * 16, 16)] = z16
        return 0
    lax.fori_loop(0, nvec, zr, 0)


def _sc_body(pred_hbm, tgt_hbm, nb_hbm, out_hbm,
             px, tx, nbuf, stats, varsep, tmp, red,
             cntv, m0v, m1v, m2v,
             nbv, outv,
             sh1, sh1g, sh2, sh2g):
    c = lax.axis_index("c")
    s = lax.axis_index("s")
    lanes = lax.iota(I32, 16)
    lanesf = lanes.astype(F32)
    z16 = jnp.zeros((16,), F32)
    ones16 = jnp.ones((16,), F32)

    pltpu.sync_copy(nb_hbm, nbv)

    def one_batch(bl, _):
        b = c * 2 + bl
        base = s * PPT

        _zero_range(stats, 0, S1 // 16, z16)
        _zero_range(varsep, 0, S2 // 16, z16)

        for ch in range(3):
            hoff = (b * 3 + ch) * NPIX + base
            pltpu.sync_copy(pred_hbm.at[pl.ds(hoff, PPT)],
                            px.at[pl.ds(ch * PPT, PPT)])
            pltpu.sync_copy(tgt_hbm.at[pl.ds(hoff, PPT)],
                            tx.at[pl.ds(ch * PPT, PPT)])

        def p1(i):
            sl0 = pl.ds(i * 16, 16)
            sl1 = pl.ds(PPT + i * 16, 16)
            sl2 = pl.ds(2 * PPT + i * 16, 16)
            labv = tx[sl0] * 9 + tx[sl1] * 3 + tx[sl2]
            addr = lanes * LPAD + labv
            plsc.addupdate_scatter(stats, [addr], ones16)
            plsc.addupdate_scatter(stats, [addr + 512], px[sl0])
            plsc.addupdate_scatter(stats, [addr + 1024], px[sl1])
            plsc.addupdate_scatter(stats, [addr + 1536], px[sl2])
        plsc.parallel_loop(0, NVEC, 1, unroll=4)(p1)

        pltpu.sync_copy(stats, sh1.at[pl.ds(s * S1, S1)])
        plsc.subcore_barrier()
        myco = s * 128
        for ts in range(NTILE):
            pltpu.sync_copy(sh1.at[pl.ds(ts * S1 + myco, 128)],
                            tmp.at[pl.ds(ts * 128, 128)])
        for j in range(8):
            acc = z16
            for ts in range(NTILE):
                acc = acc + tmp[pl.ds(ts * 128 + j * 16, 16)]
            red[pl.ds(j * 16, 16)] = acc
        pltpu.sync_copy(red.at[pl.ds(0, 128)], sh1g.at[pl.ds(s * 128, 128)])
        plsc.subcore_barrier()
        pltpu.sync_copy(sh1g, stats)

        for k in range(2):
            ksl = pl.ds(k * 16, 16)
            cnt = z16
            s0 = z16
            s1 = z16
            s2 = z16
            for l in range(16):
                off = l * LPAD + k * 16
                cnt = cnt + stats[pl.ds(off, 16)]
                s0 = s0 + stats[pl.ds(512 + off, 16)]
                s1 = s1 + stats[pl.ds(1024 + off, 16)]
                s2 = s2 + stats[pl.ds(1536 + off, 16)]
            cm = jnp.maximum(cnt, 1.0)
            m0 = s0 / cm
            m1 = s1 / cm
            m2 = s2 / cm
            if k == 0:
                bgz = jnp.where(lanesf == 0.0, 0.0, 1.0)
                m0 = m0 * bgz
                m1 = m1 * bgz
                m2 = m2 * bgz
            cntv[ksl] = cnt
            m0v[ksl] = m0
            m1v[ksl] = m1
            m2v[ksl] = m2

        def p2(i):
            sl0 = pl.ds(i * 16, 16)
            sl1 = pl.ds(PPT + i * 16, 16)
            sl2 = pl.ds(2 * PPT + i * 16, 16)
            labv = tx[sl0] * 9 + tx[sl1] * 3 + tx[sl2]
            addr = lanes * LPAD + labv
            x0 = px[sl0]
            x1 = px[sl1]
            x2 = px[sl2]
            nbuf[sl0] = x0 * x0 + x1 * x1 + x2 * x2
            d0 = x0 - plsc.load_gather(m0v, [labv])
            d1 = x1 - plsc.load_gather(m1v, [labv])
            d2 = x2 - plsc.load_gather(m2v, [labv])
            h = z16
            for d in (d0, d1, d2):
                ad = jnp.abs(d)
                h = h + jnp.where(ad <= 1.0, 0.5 * d * d, ad - 0.5)
            plsc.addupdate_scatter(varsep, [addr], h)
            so = 300.0 / (1.0 + (d0 * d0 + d1 * d1 + d2 * d2))
            plsc.addupdate_scatter(varsep, [addr + 512], so)
        plsc.parallel_loop(0, NVEC, 1, unroll=4)(p2)

        mck = [(m0v[pl.ds(k * 16, 16)], m1v[pl.ds(k * 16, 16)],
                m2v[pl.ds(k * 16, 16)]) for k in range(2)]
        a0c = [-2.0 * m for (m, _, _) in mck]
        a1c = [-2.0 * m for (_, m, _) in mck]
        a2c = [-2.0 * m for (_, _, m) in mck]
        ccc = [1.0 + m0 * m0 + m1 * m1 + m2 * m2 for (m0, m1, m2) in mck]
        GBLK = 4
        for g0 in range(1, NLAB, GBLK):
            gs = list(range(g0, min(g0 + GBLK, NLAB)))
            co = [(a0c[g // 16][g % 16], a1c[g // 16][g % 16],
                   a2c[g // 16][g % 16], ccc[g // 16][g % 16]) for g in gs]

            def pg(i, accs, co=co):
                accs = list(accs)
                x0 = px[pl.ds(i * 16, 16)]
                x1 = px[pl.ds(PPT + i * 16, 16)]
                x2 = px[pl.ds(2 * PPT + i * 16, 16)]
                n = nbuf[pl.ds(i * 16, 16)]
                for j, (a0, a1, a2, cg) in enumerate(co):
                    t = (n + cg) + (a0 * x0 + a1 * x1 + a2 * x2)
                    accs[j] = accs[j] + 300.0 / t
                return tuple(accs)
            accs = lax.fori_loop(0, NVEC, pg, (z16,) * len(gs))
            for g, acc in zip(gs, accs):
                sg = jnp.sum(acc)
                off = 1024 + (g // 16) * 16
                contrib = jnp.where(lanes == (g % 16), sg, 0.0)
                varsep[pl.ds(off, 16)] = varsep[pl.ds(off, 16)] + contrib

        pltpu.sync_copy(varsep, sh2.at[pl.ds(s * S2, S2)])
        plsc.subcore_barrier()
        myc2 = s * 80
        for ts in range(NTILE):
            pltpu.sync_copy(sh2.at[pl.ds(ts * S2 + myc2, 80)],
                            tmp.at[pl.ds(ts * 80, 80)])
        for j in range(5):
            acc = z16
            for ts in range(NTILE):
                acc = acc + tmp[pl.ds(ts * 80 + j * 16, 16)]
            red[pl.ds(j * 16, 16)] = acc
        pltpu.sync_copy(red.at[pl.ds(0, 80)], sh2g.at[pl.ds(s * 80, 80)])
        plsc.subcore_barrier()

        @pl.when(s == 0)
        def _():
            pltpu.sync_copy(sh2g, varsep.at[pl.ds(0, S2)])
            alv = plsc.load_gather(nbv, [jnp.full((16,), 0, I32) + b])
            lv = z16
            ctv = z16
            presk = [None, None]
            for k in range(2):
                ksl = pl.ds(k * 16, 16)
                varsum = z16
                sepown = z16
                for l in range(16):
                    off = l * LPAD + k * 16
                    varsum = varsum + varsep[pl.ds(off, 16)]
                    sepown = sepown + varsep[pl.ds(512 + off, 16)]
                cv = cntv[ksl]
                cm = jnp.maximum(cv, 1.0)
                mpos = jnp.where(cv > 0.0, 1.0, 0.0)
                if k == 0:
                    pres = mpos * jnp.where(lanesf == 0.0, alv, 1.0)
                    gz = jnp.where(lanesf > 0.0, 1.0, 0.0)
                else:
                    pres = mpos
                    gz = ones16
                presk[k] = pres
                termA = pres * varsum / (3.0 * cm)
                other = float(NPIX) - cv
                op = jnp.where(other > 0.0, 1.0, 0.0)
                wb = 10.0 * _rsqrt_nr(cm)
                sepall = varsep[pl.ds(1024 + k * 16, 16)]
                termB = (mpos * gz * op * (sepall - sepown)
                         / jnp.maximum(other, 1.0) * wb)
                lv = lv + termA + termB
                ctv = ctv + pres

            mfk = [(m0v[pl.ds(k * 16, 16)], m1v[pl.ds(k * 16, 16)],
                    m2v[pl.ds(k * 16, 16)]) for k in range(2)]
            pacc = z16
            nacc = z16
            for i in range(NLAB):
                mi = mfk[i // 16]
                pi = presk[i // 16][i % 16]
                m0i = mi[0][i % 16]
                m1i = mi[1][i % 16]
                m2i = mi[2][i % 16]
                for k in range(2):
                    jv = lanesf + float(k * 16)
                    d0 = mfk[k][0] - m0i
                    d1 = mfk[k][1] - m1i
                    d2 = mfk[k][2] - m2i
                    sq = d0 * d0 + d1 * d1 + d2 * d2
                    mk = jnp.where(jv > float(i), 1.0, 0.0) * presk[k] * pi
                    pacc = pacc + 300.0 / (sq + 1.0) * mk
                    nacc = nacc + mk
            spn = z16 + jnp.sum(nacc)
            spt = (z16 + jnp.sum(pacc)) / jnp.maximum(spn, 1.0)
            lossv = (z16 + jnp.sum(lv)) + jnp.where(spn > 0.0, spt, z16)
            outvec = lossv / jnp.maximum(z16 + jnp.sum(ctv), 1.0)
            outv[pl.ds(0, 16)] = jnp.where(lanes == 0, outvec, z16)
            pltpu.sync_copy(outv, out_hbm.at[pl.ds(b * 16, 16)])
        plsc.subcore_barrier()
        return 0

    lax.fori_loop(0, 2, one_batch, 0)


@jax.jit
def _distance_loss(pred, tgt, nb):
    mesh = plsc.VectorSubcoreMesh(core_axis_name="c", subcore_axis_name="s",
                                  num_cores=2, num_subcores=16)
    run = pl.kernel(
        _sc_body,
        out_type=jax.ShapeDtypeStruct((64,), F32),
        mesh=mesh,
        compiler_params=pltpu.CompilerParams(
            needs_layout_passes=False, use_tc_tiling_on_sc=False),
        scratch_types=[
            pltpu.VMEM((3 * PPT,), F32),
            pltpu.VMEM((3 * PPT,), I32),
            pltpu.VMEM((PPT,), F32),
            pltpu.VMEM((S1,), F32),
            pltpu.VMEM((S2,), F32),
            pltpu.VMEM((S1,), F32),
            pltpu.VMEM((128,), F32),
            pltpu.VMEM((LPAD,), F32),
            pltpu.VMEM((LPAD,), F32),
            pltpu.VMEM((LPAD,), F32),
            pltpu.VMEM((LPAD,), F32),
            pltpu.VMEM((16,), F32),
            pltpu.VMEM((16,), F32),
            pltpu.VMEM_SHARED((NTILE * S1,), F32),
            pltpu.VMEM_SHARED((S1,), F32),
            pltpu.VMEM_SHARED((NTILE * S2,), F32),
            pltpu.VMEM_SHARED((S2,), F32),
        ],
    )
    return run(pred, tgt, nb)


def kernel(prediction, target, no_bg):
    B, C, H, W = prediction.shape
    pred = prediction.astype(F32).reshape(-1)
    tgt = target.astype(I32).reshape(-1)
    nb = jnp.zeros((16,), F32).at[:B].set(1.0 - no_bg.astype(F32))
    out = _distance_loss(pred, tgt, nb)
    return jnp.sum(out.reshape(B, 16)[:, 0]) / float(B)
